# baseline (speedup 1.0000x reference)
import functools

import jax
import jax.numpy as jnp
from jax import lax
from jax.experimental import pallas as pl
from jax.experimental.pallas import tpu as pltpu
from jax.experimental.pallas import tpu_sc as plsc

_NUM_CORES = 2
_NUM_SUBCORES = 16
_NUM_WORKERS = _NUM_CORES * _NUM_SUBCORES
_LANES = 16


@functools.partial(jax.jit, static_argnames=("batch", "seq_len", "rows_per_chunk"))
def _embed_sc(ids_flat, wte, wpe, *, batch, seq_len, rows_per_chunk):
    n_tok = ids_flat.shape[0]
    n_embd = wte.shape[1]
    s_per_worker = seq_len // _NUM_WORKERS
    k = rows_per_chunk
    halves = s_per_worker // k
    n_chunks = batch * halves
    lanes_per_row = n_embd // _LANES

    mesh = plsc.VectorSubcoreMesh(
        core_axis_name="c",
        subcore_axis_name="s",
        num_cores=_NUM_CORES,
        num_subcores=_NUM_SUBCORES,
    )

    @functools.partial(
        pl.kernel,
        out_type=jax.ShapeDtypeStruct((n_tok, n_embd), jnp.float32),
        mesh=mesh,
        scratch_types=[
            pltpu.VMEM((2, k), jnp.int32),
            pltpu.VMEM((2, k, n_embd), jnp.float32),
            pltpu.VMEM((s_per_worker, n_embd), jnp.float32),
            pltpu.SemaphoreType.DMA,
            pltpu.SemaphoreType.DMA,
            pltpu.SemaphoreType.DMA,
            pltpu.SemaphoreType.DMA,
        ],
    )
    def body(ids_hbm, wte_hbm, wpe_hbm, out_hbm, idx_v, rows_v, wpe_v,
             sem_g0, sem_g1, sem_s0, sem_s1):
        wid = lax.axis_index("s") * _NUM_CORES + lax.axis_index("c")
        s0 = wid * s_per_worker
        sem_g = (sem_g0, sem_g1)
        sem_s = (sem_s0, sem_s1)

        wpe_load = pltpu.async_copy(
            wpe_hbm.at[pl.ds(s0, s_per_worker), :], wpe_v, sem_s0
        )

        def flat_base(c):
            b, h = divmod(c, halves)
            return b * seq_len + s0 + h * k

        def start_gather(c):
            bi = c % 2
            pltpu.sync_copy(ids_hbm.at[pl.ds(flat_base(c), k)], idx_v.at[bi])
            return pltpu.async_copy(wte_hbm.at[idx_v.at[bi]], rows_v.at[bi], sem_g[bi])

        gathers = [None] * n_chunks
        stores = [None] * n_chunks
        gathers[0] = start_gather(0)
        wpe_load.wait()
        for c in range(n_chunks):
            bi = c % 2
            if c + 1 < n_chunks:
                if c >= 1:
                    stores[c - 1].wait()
                gathers[c + 1] = start_gather(c + 1)
            gathers[c].wait()

            woff = (c % halves) * k

            def add_row(r):
                for j in range(lanes_per_row):
                    sl = pl.ds(j * _LANES, _LANES)
                    rows_v[bi, r, sl] += wpe_v[woff + r, sl]

            pl.loop(0, k)(add_row)
            stores[c] = pltpu.async_copy(
                rows_v.at[bi], out_hbm.at[pl.ds(flat_base(c), k), :], sem_s[bi]
            )
        stores[n_chunks - 2].wait()
        stores[n_chunks - 1].wait()

    return body(ids_flat, wte, wpe)


def kernel(input_ids, wte, wpe):
    batch, seq_len = input_ids.shape
    out = _embed_sc(
        input_ids.reshape(-1), wte, wpe,
        batch=batch, seq_len=seq_len, rows_per_chunk=32,
    )
    return out.reshape(batch, seq_len, wte.shape[1])


# speedup vs baseline: 1.0517x; 1.0517x over previous
import functools

import jax
import jax.numpy as jnp
from jax import lax
from jax.experimental import pallas as pl
from jax.experimental.pallas import tpu as pltpu
from jax.experimental.pallas import tpu_sc as plsc

_NUM_CORES = 2
_NUM_SUBCORES = 16
_NUM_WORKERS = _NUM_CORES * _NUM_SUBCORES
_LANES = 16


@functools.partial(jax.jit, static_argnames=("seq_len", "rows_per_chunk"))
def _embed_sc(ids_flat, wte, wpe, *, seq_len, rows_per_chunk):
    n_tok = ids_flat.shape[0]
    n_embd = wte.shape[1]
    rows_per_worker = n_tok // _NUM_WORKERS
    k = rows_per_chunk
    n_chunks = rows_per_worker // k

    mesh = plsc.VectorSubcoreMesh(
        core_axis_name="c",
        subcore_axis_name="s",
        num_cores=_NUM_CORES,
        num_subcores=_NUM_SUBCORES,
    )

    @functools.partial(
        pl.kernel,
        out_type=jax.ShapeDtypeStruct((n_tok, n_embd), jnp.float32),
        mesh=mesh,
        scratch_types=[
            pltpu.VMEM((k,), jnp.int32),
            pltpu.VMEM((k, n_embd), jnp.float32),
            pltpu.VMEM((k, n_embd), jnp.float32),
            pltpu.SemaphoreType.DMA,
            pltpu.SemaphoreType.DMA,
        ],
    )
    def body(ids_hbm, wte_hbm, wpe_hbm, out_hbm, idx_v, rows_v, wpe_v,
             sem_g, sem_p):
        wid = lax.axis_index("s") * _NUM_CORES + lax.axis_index("c")
        lanes_per_row = n_embd // _LANES

        for c in range(n_chunks):
            base = wid * rows_per_worker + c * k
            s_start = lax.rem(base, seq_len)
            pltpu.sync_copy(ids_hbm.at[pl.ds(base, k)], idx_v)
            gather = pltpu.async_copy(wte_hbm.at[idx_v], rows_v, sem_g)
            pos = pltpu.async_copy(wpe_hbm.at[pl.ds(s_start, k), :], wpe_v, sem_p)
            gather.wait()
            pos.wait()

            def add_row(r):
                for j in range(lanes_per_row):
                    sl = pl.ds(j * _LANES, _LANES)
                    plsc.addupdate(rows_v.at[r, sl], wpe_v[r, sl])

            plsc.parallel_loop(0, k, 1, unroll=2)(add_row)
            pltpu.sync_copy(rows_v, out_hbm.at[pl.ds(base, k), :])

    return body(ids_flat, wte, wpe)


def kernel(input_ids, wte, wpe):
    batch, seq_len = input_ids.shape
    out = _embed_sc(
        input_ids.reshape(-1), wte, wpe, seq_len=seq_len, rows_per_chunk=64
    )
    return out.reshape(batch, seq_len, wte.shape[1])


# speedup vs baseline: 1.2876x; 1.2242x over previous
import functools

import jax
import jax.numpy as jnp
from jax import lax
from jax.experimental import pallas as pl
from jax.experimental.pallas import tpu as pltpu
from jax.experimental.pallas import tpu_sc as plsc

_NUM_CORES = 2
_NUM_SUBCORES = 16
_NUM_WORKERS = _NUM_CORES * _NUM_SUBCORES
_LANES = 16


@functools.partial(jax.jit, static_argnames=("batch", "seq_len"))
def _embed_sc(ids_flat, wte, wpe, *, batch, seq_len):
    n_tok = ids_flat.shape[0]
    n_embd = wte.shape[1]
    k = seq_len // _NUM_WORKERS
    lanes_per_row = n_embd // _LANES

    mesh = plsc.VectorSubcoreMesh(
        core_axis_name="c",
        subcore_axis_name="s",
        num_cores=_NUM_CORES,
        num_subcores=_NUM_SUBCORES,
    )

    @functools.partial(
        pl.kernel,
        out_type=jax.ShapeDtypeStruct((n_tok, n_embd), jnp.float32),
        mesh=mesh,
        scratch_types=[
            pltpu.VMEM((batch * k,), jnp.int32),
            pltpu.VMEM((k, n_embd), jnp.float32),
            pltpu.VMEM((k, n_embd), jnp.float32),
            pltpu.SemaphoreType.DMA,
            pltpu.SemaphoreType.DMA,
        ],
    )
    def body(ids_hbm, wte_hbm, wpe_hbm, out_hbm, idx_v, rows_v, wpe_v,
             sem_g, sem_p):
        wid = lax.axis_index("s") * _NUM_CORES + lax.axis_index("c")
        s0 = wid * k

        wpe_load = pltpu.async_copy(wpe_hbm.at[pl.ds(s0, k), :], wpe_v, sem_p)
        id_loads = [
            pltpu.async_copy(
                ids_hbm.at[pl.ds(b * seq_len + s0, k)],
                idx_v.at[pl.ds(b * k, k)],
                sem_g,
            )
            for b in range(batch)
        ]
        for ld in id_loads:
            ld.wait()
        wpe_load.wait()

        for b in range(batch):
            base = b * seq_len + s0
            gather = pltpu.async_copy(
                wte_hbm.at[idx_v.at[pl.ds(b * k, k)]], rows_v, sem_g
            )
            gather.wait()

            def add_row(r):
                for j in range(lanes_per_row):
                    sl = pl.ds(j * _LANES, _LANES)
                    rows_v[r, sl] += wpe_v[r, sl]

            pl.loop(0, k)(add_row)
            pltpu.sync_copy(rows_v, out_hbm.at[pl.ds(base, k), :])

    return body(ids_flat, wte, wpe)


def kernel(input_ids, wte, wpe):
    batch, seq_len = input_ids.shape
    out = _embed_sc(input_ids.reshape(-1), wte, wpe, batch=batch, seq_len=seq_len)
    return out.reshape(batch, seq_len, wte.shape[1])
